# Initial kernel scaffold; baseline (speedup 1.0000x reference)
import jax
import jax.numpy as jnp
from jax.experimental import pallas as pl


def kernel(z, codebook):
    raise NotImplementedError("write your pallas kernel here")


# speedup vs baseline: 1.7697x; 1.7697x over previous
import functools

import jax
import jax.numpy as jnp
from jax.experimental import pallas as pl

NUM_CODE = 1024
DIM_CODE = 256
TOK_BLOCK = 1024


def _vq_body(nblocks, total, z_ref, zn_ref, cn_ref, cb_ref,
             zq_ref, idx_ref, loss_ref, res_ref):
    i = pl.program_id(0)
    z = z_ref[...]
    cb = cb_ref[...]
    zn = zn_ref[...]
    cn = cn_ref[...]
    d = (zn + cn) - 2.0 * jnp.dot(
        z, cb.T, preferred_element_type=jnp.float32)
    idx = jnp.argmin(d, axis=1).astype(jnp.int32)
    onehot = (idx[:, None] == jax.lax.broadcasted_iota(
        jnp.int32, (z.shape[0], NUM_CODE), 1)).astype(jnp.float32)
    zq = jnp.dot(onehot, cb, preferred_element_type=jnp.float32)
    r = z - zq
    zq_ref[...] = zq
    idx_ref[0, 0, :] = idx
    res_ref[...] = r
    part = jnp.sum(r * r, keepdims=True).reshape(1, 1)

    @pl.when(i == 0)
    def _():
        loss_ref[...] = jnp.zeros((1, 1), jnp.float32)

    loss_ref[...] += part

    @pl.when(i == nblocks - 1)
    def _():
        loss_ref[...] = loss_ref[...] * (2.0 / total)


@jax.jit
def kernel(z, codebook):
    B, L, E = z.shape
    n_tok = B * L
    z_flat = z.reshape(n_tok, E)
    nb = n_tok // TOK_BLOCK

    znorm = jnp.sum(z_flat ** 2, axis=1, keepdims=True)
    cnorm = jnp.sum(codebook ** 2, axis=1)[None, :]

    zq, idx, loss, res = pl.pallas_call(
        functools.partial(_vq_body, nb, z.size),
        grid=(nb,),
        in_specs=[
            pl.BlockSpec((TOK_BLOCK, E), lambda i: (i, 0)),
            pl.BlockSpec((TOK_BLOCK, 1), lambda i: (i, 0)),
            pl.BlockSpec((1, NUM_CODE), lambda i: (0, 0)),
            pl.BlockSpec((NUM_CODE, E), lambda i: (0, 0)),
        ],
        out_specs=[
            pl.BlockSpec((TOK_BLOCK, E), lambda i: (i, 0)),
            pl.BlockSpec((1, 1, TOK_BLOCK), lambda i: (i, 0, 0)),
            pl.BlockSpec((1, 1), lambda i: (0, 0)),
            pl.BlockSpec((TOK_BLOCK, E), lambda i: (i, 0)),
        ],
        out_shape=[
            jax.ShapeDtypeStruct((n_tok, E), jnp.float32),
            jax.ShapeDtypeStruct((nb, 1, TOK_BLOCK), jnp.int32),
            jax.ShapeDtypeStruct((1, 1), jnp.float32),
            jax.ShapeDtypeStruct((n_tok, E), jnp.float32),
        ],
    )(z_flat, znorm, cnorm, codebook)

    return (zq.reshape(B, L, E), idx.reshape(B, L), loss[0, 0],
            res.reshape(B, L, E))
